# baseline (speedup 1.0000x reference)
import functools

import jax
import jax.numpy as jnp
from jax import lax
from jax.experimental import pallas as pl
from jax.experimental.pallas import tpu as pltpu
from jax.experimental.pallas import tpu_sc as plsc

B = 4096
D = 1024
K = 8192
S = 16

KT = 512
NK = K // KT

NW = 32
RPW = B // NW
CH = 32
NCH = RPW // CH


def _prep_body(w_ref, wb_ref, wt_ref):
    w = w_ref[...]
    wb_ref[...] = w.astype(jnp.bfloat16)
    wt_ref[...] = w.T


def _prep(W):
    return pl.pallas_call(
        _prep_body,
        grid=(NK,),
        in_specs=[pl.BlockSpec((D, KT), lambda k: (0, k))],
        out_specs=[
            pl.BlockSpec((D, KT), lambda k: (0, k)),
            pl.BlockSpec((KT, D), lambda k: (k, 0)),
        ],
        out_shape=[
            jax.ShapeDtypeStruct((D, K), jnp.bfloat16),
            jax.ShapeDtypeStruct((K, D), jnp.float32),
        ],
    )(W)


BH = B // 2


def _step_body(r_ref, wb_ref, j_ref, z_ref, rb_scr, c_scr, smax_scr,
               sidx_scr, sval_scr):
    k = pl.program_id(1)

    @pl.when(k == 0)
    def _():
        rb_scr[...] = r_ref[...].astype(jnp.bfloat16)
        smax_scr[...] = jnp.full((BH, 1), -1.0, jnp.float32)
        sidx_scr[...] = jnp.zeros((BH, 1), jnp.int32)
        sval_scr[...] = jnp.zeros((BH, 1), jnp.float32)

    @pl.when(k < NK)
    def _():
        c_scr[k % 2] = jnp.dot(rb_scr[...], wb_ref[...],
                               preferred_element_type=jnp.float32)

    @pl.when(k > 0)
    def _():
        c = c_scr[(k - 1) % 2]
        a = jnp.abs(c)
        lmax = jnp.max(a, axis=1, keepdims=True)
        ii = lax.broadcasted_iota(jnp.int32, (BH, KT), 1)
        lidx = jnp.min(jnp.where(a == lmax, ii, KT), axis=1, keepdims=True)
        lval = jnp.sum(jnp.where(ii == lidx, c, 0.0), axis=1, keepdims=True)

        upd = lmax > smax_scr[...]
        smax_scr[...] = jnp.where(upd, lmax, smax_scr[...])
        sidx_scr[...] = jnp.where(upd, lidx + (k - 1) * KT, sidx_scr[...])
        sval_scr[...] = jnp.where(upd, lval, sval_scr[...])

    @pl.when(k == NK)
    def _():
        j_ref[...] = sidx_scr[...]
        z_ref[...] = sval_scr[...]


def _tc_step(r, Wb):
    return pl.pallas_call(
        _step_body,
        grid=(2, NK + 1),
        in_specs=[
            pl.BlockSpec((BH, D), lambda b, k: (b, 0)),
            pl.BlockSpec((D, KT), lambda b, k: (0, jnp.minimum(k, NK - 1))),
        ],
        out_specs=[
            pl.BlockSpec((BH, 1), lambda b, k: (b, 0)),
            pl.BlockSpec((BH, 1), lambda b, k: (b, 0)),
        ],
        out_shape=[
            jax.ShapeDtypeStruct((B, 1), jnp.int32),
            jax.ShapeDtypeStruct((B, 1), jnp.float32),
        ],
        scratch_shapes=[
            pltpu.VMEM((BH, D), jnp.bfloat16),
            pltpu.VMEM((2, BH, KT), jnp.float32),
            pltpu.VMEM((BH, 1), jnp.float32),
            pltpu.VMEM((BH, 1), jnp.int32),
            pltpu.VMEM((BH, 1), jnp.float32),
        ],
    )(r, Wb)


def _chunk_update(z_c, sel_v, r_v):
    for g in range(CH // 16):
        zvec = z_c[pl.ds(g * 16, 16)]
        for rr in range(16):
            i = g * 16 + rr
            zz = zvec[rr]

            def col_body(t, _, i=i, zz=zz):
                off = t * 64
                for u in range(4):
                    sl = pl.ds(off + u * 16, 16)
                    r_v[i, sl] = r_v[i, sl] - zz * sel_v[i, sl]
                return 0

            lax.fori_loop(0, D // 64, col_body, 0)


def _sc_update_body(j_hbm, z_hbm, r_hbm, wt_hbm, out_hbm,
                    idx_c, z_c, sel_v, r_v, sem):
    wid = lax.axis_index("s") * 2 + lax.axis_index("c")
    base = wid * RPW

    def chunk_body(ci, _):
        rows0 = base + ci * CH
        pltpu.sync_copy(j_hbm.at[pl.ds(rows0, CH)], idx_c)
        pltpu.async_copy(wt_hbm.at[idx_c], sel_v, sem).wait()
        pltpu.sync_copy(z_hbm.at[pl.ds(rows0, CH)], z_c)
        pltpu.sync_copy(r_hbm.at[pl.ds(rows0, CH)], r_v)
        _chunk_update(z_c, sel_v, r_v)
        pltpu.sync_copy(r_v, out_hbm.at[pl.ds(rows0, CH)])
        return 0

    lax.fori_loop(0, NCH, chunk_body, 0)


def _sc_update(j, z, r, WT):
    mesh = plsc.VectorSubcoreMesh(core_axis_name="c", subcore_axis_name="s",
                                  num_cores=2, num_subcores=16)
    fn = pl.kernel(
        _sc_update_body,
        out_type=jax.ShapeDtypeStruct((B, D), jnp.float32),
        mesh=mesh,
        scratch_types=[
            pltpu.VMEM((CH,), jnp.int32),
            pltpu.VMEM((CH,), jnp.float32),
            pltpu.VMEM((CH, D), jnp.float32),
            pltpu.VMEM((CH, D), jnp.float32),
            pltpu.SemaphoreType.DMA,
        ],
    )
    return fn(j, z, r, WT)


def _sc_final_body(j_hbm, z_hbm, r_hbm, x_hbm, wt_hbm, out_hbm,
                   idx_c, z_c, sel_v, r_v, x_v, sem):
    wid = lax.axis_index("s") * 2 + lax.axis_index("c")
    base = wid * RPW

    def chunk_body(ci, _):
        rows0 = base + ci * CH
        pltpu.sync_copy(j_hbm.at[pl.ds(rows0, CH)], idx_c)
        pltpu.async_copy(wt_hbm.at[idx_c], sel_v, sem).wait()
        pltpu.sync_copy(z_hbm.at[pl.ds(rows0, CH)], z_c)
        pltpu.sync_copy(r_hbm.at[pl.ds(rows0, CH)], r_v)
        pltpu.sync_copy(x_hbm.at[pl.ds(rows0, CH)], x_v)

        for g in range(CH // 16):
            zvec = z_c[pl.ds(g * 16, 16)]
            for rr in range(16):
                i = g * 16 + rr
                zz = zvec[rr]

                def col_body(t, _, i=i, zz=zz):
                    off = t * 64
                    for u in range(4):
                        sl = pl.ds(off + u * 16, 16)
                        r_v[i, sl] = x_v[i, sl] - (r_v[i, sl]
                                                   - zz * sel_v[i, sl])
                    return 0

                lax.fori_loop(0, D // 64, col_body, 0)

        pltpu.sync_copy(r_v, out_hbm.at[pl.ds(rows0, CH)])
        return 0

    lax.fori_loop(0, NCH, chunk_body, 0)


def _sc_final(j, z, r, x, WT):
    mesh = plsc.VectorSubcoreMesh(core_axis_name="c", subcore_axis_name="s",
                                  num_cores=2, num_subcores=16)
    fn = pl.kernel(
        _sc_final_body,
        out_type=jax.ShapeDtypeStruct((B, D), jnp.float32),
        mesh=mesh,
        scratch_types=[
            pltpu.VMEM((CH,), jnp.int32),
            pltpu.VMEM((CH,), jnp.float32),
            pltpu.VMEM((CH, D), jnp.float32),
            pltpu.VMEM((CH, D), jnp.float32),
            pltpu.VMEM((CH, D), jnp.float32),
            pltpu.SemaphoreType.DMA,
        ],
    )
    return fn(j, z, r, x, WT)


def kernel(x, W, b_dec):
    Wb, WT = _prep(W)
    r = x - b_dec
    xh = None
    for s in range(S):
        j, z = _tc_step(r, Wb)
        jf = j.reshape(B)
        zf = z.reshape(B)
        if s < S - 1:
            r = _sc_update(jf, zf, r, WT)
        else:
            xh = _sc_final(jf, zf, r, x, WT)
    return xh


# speedup vs baseline: 1.1209x; 1.1209x over previous
import functools

import jax
import jax.numpy as jnp
from jax import lax
from jax.experimental import pallas as pl
from jax.experimental.pallas import tpu as pltpu
from jax.experimental.pallas import tpu_sc as plsc

B = 4096
D = 1024
K = 8192
S = 16

KT = 512
NK = K // KT

NW = 32
CH = 32

_MESH = dict(core_axis_name="c", subcore_axis_name="s",
             num_cores=2, num_subcores=16)


def _prep_body(w_ref, wb_ref, wt_ref):
    w = w_ref[...]
    wb_ref[...] = w.astype(jnp.bfloat16)
    wt_ref[...] = w.T


def _prep(W):
    return pl.pallas_call(
        _prep_body,
        grid=(NK,),
        in_specs=[pl.BlockSpec((D, KT), lambda k: (0, k))],
        out_specs=[
            pl.BlockSpec((D, KT), lambda k: (0, k)),
            pl.BlockSpec((KT, D), lambda k: (k, 0)),
        ],
        out_shape=[
            jax.ShapeDtypeStruct((D, K), jnp.bfloat16),
            jax.ShapeDtypeStruct((K, D), jnp.float32),
        ],
    )(W)


def _tc_step(r, Wb):
    nb = r.shape[0]

    def body(r_ref, wb_ref, j_ref, z_ref, rb_scr, c_scr, smax_scr,
             sidx_scr, sval_scr):
        k = pl.program_id(0)

        @pl.when(k == 0)
        def _():
            rb_scr[...] = r_ref[...].astype(jnp.bfloat16)
            smax_scr[...] = jnp.full((nb, 1), -1.0, jnp.float32)
            sidx_scr[...] = jnp.zeros((nb, 1), jnp.int32)
            sval_scr[...] = jnp.zeros((nb, 1), jnp.float32)

        c_scr[k % 2] = jnp.dot(rb_scr[...], wb_ref[...],
                               preferred_element_type=jnp.float32)

        c = c_scr[(k + 1) % 2]
        a = jnp.abs(c)
        lmax = jnp.max(a, axis=1, keepdims=True)
        ii = lax.broadcasted_iota(jnp.int32, (nb, KT), 1)
        lidx = jnp.min(jnp.where(a == lmax, ii, KT), axis=1, keepdims=True)
        lval = jnp.sum(jnp.where(ii == lidx, c, 0.0), axis=1, keepdims=True)

        upd = jnp.logical_and(lmax > smax_scr[...], k > 0)
        smax_scr[...] = jnp.where(upd, lmax, smax_scr[...])
        sidx_scr[...] = jnp.where(upd, lidx + (k - 1) * KT, sidx_scr[...])
        sval_scr[...] = jnp.where(upd, lval, sval_scr[...])

        @pl.when(k == NK)
        def _():
            j_ref[...] = sidx_scr[...]
            z_ref[...] = sval_scr[...]

    return pl.pallas_call(
        body,
        grid=(NK + 1,),
        in_specs=[
            pl.BlockSpec((nb, D), lambda k: (0, 0)),
            pl.BlockSpec((D, KT), lambda k: (0, jnp.minimum(k, NK - 1))),
        ],
        out_specs=[
            pl.BlockSpec((nb, 1), lambda k: (0, 0)),
            pl.BlockSpec((nb, 1), lambda k: (0, 0)),
        ],
        out_shape=[
            jax.ShapeDtypeStruct((nb, 1), jnp.int32),
            jax.ShapeDtypeStruct((nb, 1), jnp.float32),
        ],
        scratch_shapes=[
            pltpu.VMEM((nb, D), jnp.bfloat16),
            pltpu.VMEM((2, nb, KT), jnp.float32),
            pltpu.VMEM((nb, 1), jnp.float32),
            pltpu.VMEM((nb, 1), jnp.int32),
            pltpu.VMEM((nb, 1), jnp.float32),
        ],
    )(r, Wb)


def _chunk_update(z_c, sel_v, r_v):
    for g in range(CH // 16):
        zvec = z_c[pl.ds(g * 16, 16)]
        for rr in range(16):
            i = g * 16 + rr
            zz = zvec[rr]

            def col_body(t, _, i=i, zz=zz):
                off = t * 64
                for u in range(4):
                    sl = pl.ds(off + u * 16, 16)
                    r_v[i, sl] = r_v[i, sl] - zz * sel_v[i, sl]
                return 0

            lax.fori_loop(0, D // 64, col_body, 0)


def _sc_update(j, z, r, WT):
    nb = r.shape[0]
    rpw = nb // NW
    nch = rpw // CH

    def body(j_hbm, z_hbm, r_hbm, wt_hbm, out_hbm,
             idx_c, z_c, sel_v, r_v, sem):
        wid = lax.axis_index("s") * 2 + lax.axis_index("c")
        base = wid * rpw

        def chunk_body(ci, _):
            rows0 = base + ci * CH
            pltpu.sync_copy(j_hbm.at[pl.ds(rows0, CH)], idx_c)
            pltpu.async_copy(wt_hbm.at[idx_c], sel_v, sem).wait()
            pltpu.sync_copy(z_hbm.at[pl.ds(rows0, CH)], z_c)
            pltpu.sync_copy(r_hbm.at[pl.ds(rows0, CH)], r_v)
            _chunk_update(z_c, sel_v, r_v)
            pltpu.sync_copy(r_v, out_hbm.at[pl.ds(rows0, CH)])
            return 0

        lax.fori_loop(0, nch, chunk_body, 0)

    fn = pl.kernel(
        body,
        out_type=jax.ShapeDtypeStruct((nb, D), jnp.float32),
        mesh=plsc.VectorSubcoreMesh(**_MESH),
        scratch_types=[
            pltpu.VMEM((CH,), jnp.int32),
            pltpu.VMEM((CH,), jnp.float32),
            pltpu.VMEM((CH, D), jnp.float32),
            pltpu.VMEM((CH, D), jnp.float32),
            pltpu.SemaphoreType.DMA,
        ],
    )
    return fn(j, z, r, WT)


def _sc_final(j, z, r, x, WT):
    nb = r.shape[0]
    rpw = nb // NW
    nch = rpw // CH

    def body(j_hbm, z_hbm, r_hbm, x_hbm, wt_hbm, out_hbm,
             idx_c, z_c, sel_v, r_v, x_v, sem):
        wid = lax.axis_index("s") * 2 + lax.axis_index("c")
        base = wid * rpw

        def chunk_body(ci, _):
            rows0 = base + ci * CH
            pltpu.sync_copy(j_hbm.at[pl.ds(rows0, CH)], idx_c)
            pltpu.async_copy(wt_hbm.at[idx_c], sel_v, sem).wait()
            pltpu.sync_copy(z_hbm.at[pl.ds(rows0, CH)], z_c)
            pltpu.sync_copy(r_hbm.at[pl.ds(rows0, CH)], r_v)
            pltpu.sync_copy(x_hbm.at[pl.ds(rows0, CH)], x_v)

            for g in range(CH // 16):
                zvec = z_c[pl.ds(g * 16, 16)]
                for rr in range(16):
                    i = g * 16 + rr
                    zz = zvec[rr]

                    def col_body(t, _, i=i, zz=zz):
                        off = t * 64
                        for u in range(4):
                            sl = pl.ds(off + u * 16, 16)
                            r_v[i, sl] = x_v[i, sl] - (r_v[i, sl]
                                                       - zz * sel_v[i, sl])
                        return 0

                    lax.fori_loop(0, D // 64, col_body, 0)

            pltpu.sync_copy(r_v, out_hbm.at[pl.ds(rows0, CH)])
            return 0

        lax.fori_loop(0, nch, chunk_body, 0)

    fn = pl.kernel(
        body,
        out_type=jax.ShapeDtypeStruct((nb, D), jnp.float32),
        mesh=plsc.VectorSubcoreMesh(**_MESH),
        scratch_types=[
            pltpu.VMEM((CH,), jnp.int32),
            pltpu.VMEM((CH,), jnp.float32),
            pltpu.VMEM((CH, D), jnp.float32),
            pltpu.VMEM((CH, D), jnp.float32),
            pltpu.VMEM((CH, D), jnp.float32),
            pltpu.SemaphoreType.DMA,
        ],
    )
    return fn(j, z, r, x, WT)


def kernel(x, W, b_dec):
    Wb, WT = _prep(W)
    r = x - b_dec
    halves = [x[:B // 2], x[B // 2:]]
    rs = [r[:B // 2], r[B // 2:]]
    xh = [None, None]
    for s in range(S):
        picks = []
        for h in range(2):
            jh, zh = _tc_step(rs[h], Wb)
            picks.append((jh.reshape(-1), zh.reshape(-1)))
        for h in range(2):
            jf, zf = picks[h]
            if s < S - 1:
                rs[h] = _sc_update(jf, zf, rs[h], WT)
            else:
                xh[h] = _sc_final(jf, zf, rs[h], halves[h], WT)
    return jnp.concatenate(xh, axis=0)
